# Initial kernel scaffold; baseline (speedup 1.0000x reference)
import jax
import jax.numpy as jnp
from jax.experimental import pallas as pl


def kernel(violation_ids, violation_embedding):
    raise NotImplementedError("write your pallas kernel here")


# speedup vs baseline: 2.4083x; 2.4083x over previous
import functools

import jax
import jax.numpy as jnp
from jax import lax
from jax.experimental import pallas as pl
from jax.experimental.pallas import tpu as pltpu
from jax.experimental.pallas import tpu_sc as plsc

NUM_CORES = 2
NUM_SUBCORES = 16
NUM_WORKERS = NUM_CORES * NUM_SUBCORES
IDX_CHUNK = 128


def _make_lookup(V, D, B):
  assert B % (NUM_WORKERS * IDX_CHUNK) == 0
  b_per_w = B // NUM_WORKERS
  n_chunks = b_per_w // IDX_CHUNK
  mesh = plsc.VectorSubcoreMesh(core_axis_name="c", subcore_axis_name="s")

  @functools.partial(
      pl.kernel,
      mesh=mesh,
      out_type=jax.ShapeDtypeStruct((B, D), jnp.float32),
      scratch_types=[
          pltpu.VMEM((n_chunks, IDX_CHUNK), jnp.int32),
          pltpu.VMEM((b_per_w, D), jnp.float32),
          pltpu.SemaphoreType.DMA,
      ],
  )
  def lookup(table_hbm, idx_hbm, out_hbm, idx_v, rows_v, sem):
    wid = lax.axis_index("s") * NUM_CORES + lax.axis_index("c")
    base = wid * b_per_w
    pltpu.sync_copy(idx_hbm.at[wid], idx_v)
    copies = []
    for j in range(n_chunks):
      copies.append(
          pltpu.async_copy(
              table_hbm.at[idx_v.at[j]],
              rows_v.at[pl.ds(j * IDX_CHUNK, IDX_CHUNK)],
              sem,
          ))
    for c in copies:
      c.wait()
    pltpu.sync_copy(rows_v, out_hbm.at[pl.ds(base, b_per_w)])

  return lookup


_B = 16384
_LOOKUP = _make_lookup(1000, 128, _B)


@jax.jit
def kernel(violation_ids, violation_embedding):
  idx = violation_ids.astype(jnp.int32).reshape(
      NUM_WORKERS, _B // NUM_WORKERS // IDX_CHUNK, IDX_CHUNK)
  return _LOOKUP(violation_embedding, idx)
